# Initial kernel scaffold; baseline (speedup 1.0000x reference)
import jax
import jax.numpy as jnp
from jax.experimental import pallas as pl


def kernel(cycle_rep_5, cycle_nodes_5, cycle_rep_6, cycle_nodes_6, path_rep_4, path_nodes_4, path_rep_5, path_nodes_5, path_rep_6, path_nodes_6, path_rep_7, path_nodes_7, W_c5, b_c5, W_c6, b_c6, W_p4, b_p4, W_p5, b_p5, W_p6, b_p6, W_p7, b_p7):
    raise NotImplementedError("write your pallas kernel here")


# speedup vs baseline: 5.2416x; 5.2416x over previous
import functools

import jax
import jax.numpy as jnp
from jax import lax
from jax.experimental import pallas as pl
from jax.experimental.pallas import tpu as pltpu
from jax.experimental.pallas import tpu_sc as plsc

N_NODES = 50000
H = 128
FC = 32
GB = 4
ROWS_BLK = 128
GROUP_ROWS = GB * ROWS_BLK
TILE_PAD = 16 * GROUP_ROWS
CYC = ((5, 50000), (6, 60000))
PTH = ((4, 40000), (5, 50000), (6, 60000), (7, 70000))
NODES_PER_TILE = N_NODES // 16


def _padded(r):
    return ((r + TILE_PAD - 1) // TILE_PAD) * TILE_PAD


def _sc_transfer(c5r, c6r, p4r, p5r, p6r, p7r,
                 c5i, c6i, p4i, p5i, p6i, p7i, zeros_hbm,
                 g_c5, g_c6, g_p4, g_p5, g_p6, g_p7,
                 table, idx_g, data_g):
    cid = lax.axis_index("c")
    sid = lax.axis_index("s")

    cyc = ((c5r, c5i, 50000, g_c5), (c6r, c6i, 60000, g_c6))
    pth = ((p4r, p4i, 40000, g_p4), (p5r, p5i, 50000, g_p5),
           (p6r, p6i, 60000, g_p6), (p7r, p7i, 70000, g_p7))

    for srcs, dsts in ((cyc, pth), (pth, cyc)):
        @pl.loop(0, 2)
        def _u_loop(u):
            col = (2 * cid + u) * FC
            zbase = sid * NODES_PER_TILE
            pltpu.sync_copy(zeros_hbm.at[pl.ds(0, GROUP_ROWS)], data_g)
            for t in range(6):
                pltpu.sync_copy(data_g,
                                table.at[pl.ds(zbase + t * GROUP_ROWS,
                                               GROUP_ROWS)])
            pltpu.sync_copy(data_g.at[pl.ds(0, NODES_PER_TILE - 3072)],
                            table.at[pl.ds(zbase + 3072, NODES_PER_TILE - 3072)])
            plsc.subcore_barrier()

            for rep, idxp, R, _ in srcs:
                NBT = idxp.shape[0] // 16
                JP = R // ROWS_BLK
                rem = R % ROWS_BLK
                JPmax = JP if rem else JP - 1

                @pl.loop(0, NBT // GB)
                def _grp(g, _sid=sid, _col=col, _rep=rep, _idxp=idxp,
                         _NBT=NBT, _JP=JP, _rem=rem, _JPmax=JPmax):
                    j0 = _sid * _NBT + g * GB

                    @pl.when(j0 <= _JPmax)
                    def _():
                        pltpu.sync_copy(_idxp.at[pl.ds(j0, GB)], idx_g)

                        @pl.when(j0 + GB <= _JP)
                        def _():
                            pltpu.sync_copy(
                                _rep.at[pl.ds(j0 * ROWS_BLK, GROUP_ROWS),
                                        pl.ds(_col, FC)], data_g)

                        @pl.when(j0 + GB > _JP)
                        def _():
                            for t in range(GB):
                                j = j0 + t

                                @pl.when(j < _JP)
                                def _(t=t, j=j):
                                    pltpu.sync_copy(
                                        _rep.at[pl.ds(j * ROWS_BLK, ROWS_BLK),
                                                pl.ds(_col, FC)],
                                        data_g.at[pl.ds(t * ROWS_BLK, ROWS_BLK)])
                                if _rem:
                                    @pl.when(j == _JP)
                                    def _(t=t):
                                        pltpu.sync_copy(
                                            zeros_hbm.at[pl.ds(0, ROWS_BLK)],
                                            data_g.at[pl.ds(t * ROWS_BLK, ROWS_BLK)])
                                        pltpu.sync_copy(
                                            _rep.at[pl.ds(_JP * ROWS_BLK, _rem),
                                                    pl.ds(_col, FC)],
                                            data_g.at[pl.ds(t * ROWS_BLK, _rem)])

                        for t in range(GB):
                            j = j0 + t

                            @pl.when(j <= _JPmax)
                            def _(t=t):
                                pltpu.sync_copy(
                                    data_g.at[pl.ds(t * ROWS_BLK, ROWS_BLK)],
                                    table.at[idx_g.at[t]], add=True)

            plsc.subcore_barrier()

            for _, idxp, R, gout in dsts:
                NBT = idxp.shape[0] // 16
                JP = R // ROWS_BLK
                rem = R % ROWS_BLK
                JPmax = JP if rem else JP - 1

                @pl.loop(0, NBT // GB)
                def _grp(g, _sid=sid, _col=col, _gout=gout, _idxp=idxp,
                         _NBT=NBT, _JP=JP, _rem=rem, _JPmax=JPmax):
                    j0 = _sid * _NBT + g * GB

                    @pl.when(j0 <= _JPmax)
                    def _():
                        pltpu.sync_copy(_idxp.at[pl.ds(j0, GB)], idx_g)
                        for t in range(GB):
                            j = j0 + t

                            @pl.when(j <= _JPmax)
                            def _(t=t):
                                pltpu.sync_copy(
                                    table.at[idx_g.at[t]],
                                    data_g.at[pl.ds(t * ROWS_BLK, ROWS_BLK)])

                        @pl.when(j0 + GB <= _JP)
                        def _():
                            pltpu.sync_copy(
                                data_g,
                                _gout.at[pl.ds(j0 * ROWS_BLK, GROUP_ROWS),
                                         pl.ds(_col, FC)])

                        @pl.when(j0 + GB > _JP)
                        def _():
                            for t in range(GB):
                                j = j0 + t

                                @pl.when(j < _JP)
                                def _(t=t, j=j):
                                    pltpu.sync_copy(
                                        data_g.at[pl.ds(t * ROWS_BLK, ROWS_BLK)],
                                        _gout.at[pl.ds(j * ROWS_BLK, ROWS_BLK),
                                                 pl.ds(_col, FC)])
                                if _rem:
                                    @pl.when(j == _JP)
                                    def _(t=t):
                                        pltpu.sync_copy(
                                            data_g.at[pl.ds(t * ROWS_BLK, _rem)],
                                            _gout.at[pl.ds(_JP * ROWS_BLK, _rem),
                                                     pl.ds(_col, FC)])

            plsc.subcore_barrier()


def _make_sc_call():
    out_type = tuple(jax.ShapeDtypeStruct((r, H), jnp.float32)
                     for _, r in CYC + PTH)
    mesh = plsc.VectorSubcoreMesh(core_axis_name="c", subcore_axis_name="s")
    return pl.kernel(
        _sc_transfer,
        out_type=out_type,
        mesh=mesh,
        compiler_params=pltpu.CompilerParams(use_tc_tiling_on_sc=False),
        scratch_types=[
            pltpu.VMEM_SHARED((N_NODES, FC), jnp.float32),
            pltpu.VMEM((GB, ROWS_BLK), jnp.int32),
            pltpu.VMEM((GROUP_ROWS, FC), jnp.float32),
        ],
    )


IB = 80


def _autobahn_body(s, g_ref, w1_ref, w0_ref, b_ref, o_ref):
    br = IB * s
    g = g_ref[...]
    out1 = jnp.dot(g, w1_ref[...], preferred_element_type=jnp.float32)
    row_inst = lax.broadcasted_iota(jnp.int32, (IB, br), 1) // s
    inst = lax.broadcasted_iota(jnp.int32, (IB, br), 0)
    m = (row_inst == inst).astype(jnp.float32)
    tsum = jnp.dot(m, g, preferred_element_type=jnp.float32)
    t0 = jnp.dot(tsum, w0_ref[...], preferred_element_type=jnp.float32)
    row_inst_t = lax.broadcasted_iota(jnp.int32, (br, IB), 0) // s
    inst_t = lax.broadcasted_iota(jnp.int32, (br, IB), 1)
    mt = (row_inst_t == inst_t).astype(jnp.float32)
    out0 = jnp.dot(mt, t0, preferred_element_type=jnp.float32)
    o_ref[...] = out1 + out0 + b_ref[...]


def _autobahn(s, rows, g, w, b):
    br = IB * s
    grid = rows // br
    w1 = w[:H]
    w0 = w[H:]
    b2 = b.reshape(1, H)
    return pl.pallas_call(
        functools.partial(_autobahn_body, s),
        grid=(grid,),
        in_specs=[
            pl.BlockSpec((br, H), lambda i: (i, 0)),
            pl.BlockSpec((H, H), lambda i: (0, 0)),
            pl.BlockSpec((H, H), lambda i: (0, 0)),
            pl.BlockSpec((1, H), lambda i: (0, 0)),
        ],
        out_specs=pl.BlockSpec((br, H), lambda i: (i, 0)),
        out_shape=jax.ShapeDtypeStruct((rows, H), jnp.float32),
        compiler_params=pltpu.CompilerParams(
            dimension_semantics=("arbitrary",)),
    )(g, w1, w0, b2)


def _pad_idx(nodes, r):
    p = _padded(r)
    pad = jnp.arange(p - r, dtype=jnp.int32) % N_NODES
    return jnp.concatenate([nodes, pad]).reshape(p // ROWS_BLK, ROWS_BLK)


def kernel(cycle_rep_5, cycle_nodes_5, cycle_rep_6, cycle_nodes_6,
           path_rep_4, path_nodes_4, path_rep_5, path_nodes_5,
           path_rep_6, path_nodes_6, path_rep_7, path_nodes_7,
           W_c5, b_c5, W_c6, b_c6,
           W_p4, b_p4, W_p5, b_p5, W_p6, b_p6, W_p7, b_p7):
    idx = [_pad_idx(n, r) for n, (_, r) in zip(
        (cycle_nodes_5, cycle_nodes_6, path_nodes_4, path_nodes_5,
         path_nodes_6, path_nodes_7), CYC + PTH)]
    zeros = jnp.zeros((1024, FC), jnp.float32)

    g_c5, g_c6, g_p4, g_p5, g_p6, g_p7 = _make_sc_call()(
        cycle_rep_5, cycle_rep_6, path_rep_4, path_rep_5, path_rep_6,
        path_rep_7, *idx, zeros)

    cycle_outs = (
        _autobahn(5, 50000, g_c5, W_c5, b_c5),
        _autobahn(6, 60000, g_c6, W_c6, b_c6),
    )
    path_outs = (
        _autobahn(4, 40000, g_p4, W_p4, b_p4),
        _autobahn(5, 50000, g_p5, W_p5, b_p5),
        _autobahn(6, 60000, g_p6, W_p6, b_p6),
        _autobahn(7, 70000, g_p7, W_p7, b_p7),
    )
    return (cycle_outs, path_outs)


# speedup vs baseline: 5.6659x; 1.0810x over previous
import functools

import jax
import jax.numpy as jnp
from jax import lax
from jax.experimental import pallas as pl
from jax.experimental.pallas import tpu as pltpu
from jax.experimental.pallas import tpu_sc as plsc

N_NODES = 50000
H = 128
FC = 32
GB = 4
ROWS_BLK = 128
GROUP_ROWS = GB * ROWS_BLK
TILE_PAD = 16 * GROUP_ROWS
CYC = ((5, 50000), (6, 60000))
PTH = ((4, 40000), (5, 50000), (6, 60000), (7, 70000))
NODES_PER_TILE = N_NODES // 16


def _padded(r):
    return ((r + TILE_PAD - 1) // TILE_PAD) * TILE_PAD


def _sc_transfer(c5r, c6r, p4r, p5r, p6r, p7r,
                 c5i, c6i, p4i, p5i, p6i, p7i, zeros_hbm,
                 g_c5, g_c6, g_p4, g_p5, g_p6, g_p7,
                 table, idx_g, data_g, fsem, ssem, gsem):
    cid = lax.axis_index("c")
    sid = lax.axis_index("s")

    cyc = ((c5r, c5i, 50000, g_c5), (c6r, c6i, 60000, g_c6))
    pth = ((p4r, p4i, 40000, g_p4), (p5r, p5i, 50000, g_p5),
           (p6r, p6i, 60000, g_p6), (p7r, p7i, 70000, g_p7))

    for srcs, dsts in ((cyc, pth), (pth, cyc)):
        @pl.loop(0, 2)
        def _u_loop(u):
            col = (2 * cid + u) * FC
            zbase = sid * NODES_PER_TILE
            pltpu.sync_copy(zeros_hbm.at[pl.ds(0, GROUP_ROWS)], data_g)
            for t in range(6):
                pltpu.sync_copy(data_g,
                                table.at[pl.ds(zbase + t * GROUP_ROWS,
                                               GROUP_ROWS)])
            pltpu.sync_copy(data_g.at[pl.ds(0, NODES_PER_TILE - 3072)],
                            table.at[pl.ds(zbase + 3072, NODES_PER_TILE - 3072)])
            plsc.subcore_barrier()

            for rep, idxp, R, _ in srcs:
                NBT = idxp.shape[0] // 16
                JP = R // ROWS_BLK
                rem = R % ROWS_BLK
                JPmax = JP if rem else JP - 1

                @pl.loop(0, NBT // GB)
                def _grp(g, _sid=sid, _col=col, _rep=rep, _idxp=idxp,
                         _NBT=NBT, _JP=JP, _rem=rem, _JPmax=JPmax):
                    j0 = _sid * _NBT + g * GB

                    @pl.when(j0 <= _JPmax)
                    def _():
                        di = pltpu.async_copy(_idxp.at[pl.ds(j0, GB)],
                                              idx_g, fsem)

                        @pl.when(j0 + GB <= _JP)
                        def _():
                            pltpu.async_copy(
                                _rep.at[pl.ds(j0 * ROWS_BLK, GROUP_ROWS),
                                        pl.ds(_col, FC)], data_g, fsem).wait()

                        @pl.when(j0 + GB > _JP)
                        def _():
                            for t in range(GB):
                                j = j0 + t

                                @pl.when(j < _JP)
                                def _(t=t, j=j):
                                    pltpu.sync_copy(
                                        _rep.at[pl.ds(j * ROWS_BLK, ROWS_BLK),
                                                pl.ds(_col, FC)],
                                        data_g.at[pl.ds(t * ROWS_BLK, ROWS_BLK)])
                                if _rem:
                                    @pl.when(j == _JP)
                                    def _(t=t):
                                        pltpu.sync_copy(
                                            zeros_hbm.at[pl.ds(0, ROWS_BLK)],
                                            data_g.at[pl.ds(t * ROWS_BLK, ROWS_BLK)])
                                        pltpu.sync_copy(
                                            _rep.at[pl.ds(_JP * ROWS_BLK, _rem),
                                                    pl.ds(_col, FC)],
                                            data_g.at[pl.ds(t * ROWS_BLK, _rem)])

                        di.wait()
                        for t in range(GB):
                            j = j0 + t

                            @pl.when(j <= _JPmax)
                            def _(t=t):
                                pltpu.async_copy(
                                    data_g.at[pl.ds(t * ROWS_BLK, ROWS_BLK)],
                                    table.at[idx_g.at[t]], ssem, add=True)
                        for t in range(GB):
                            j = j0 + t

                            @pl.when(j <= _JPmax)
                            def _(t=t):
                                pltpu.make_async_copy(
                                    zeros_hbm.at[pl.ds(0, ROWS_BLK)],
                                    data_g.at[pl.ds(t * ROWS_BLK, ROWS_BLK)],
                                    ssem).wait()

            plsc.subcore_barrier()

            for _, idxp, R, gout in dsts:
                NBT = idxp.shape[0] // 16
                JP = R // ROWS_BLK
                rem = R % ROWS_BLK
                JPmax = JP if rem else JP - 1

                @pl.loop(0, NBT // GB)
                def _grp(g, _sid=sid, _col=col, _gout=gout, _idxp=idxp,
                         _NBT=NBT, _JP=JP, _rem=rem, _JPmax=JPmax):
                    j0 = _sid * _NBT + g * GB

                    @pl.when(j0 <= _JPmax)
                    def _():
                        pltpu.async_copy(_idxp.at[pl.ds(j0, GB)],
                                         idx_g, fsem).wait()
                        for t in range(GB):
                            j = j0 + t

                            @pl.when(j <= _JPmax)
                            def _(t=t):
                                pltpu.async_copy(
                                    table.at[idx_g.at[t]],
                                    data_g.at[pl.ds(t * ROWS_BLK, ROWS_BLK)],
                                    gsem)
                        for t in range(GB):
                            j = j0 + t

                            @pl.when(j <= _JPmax)
                            def _(t=t):
                                pltpu.make_async_copy(
                                    zeros_hbm.at[pl.ds(0, ROWS_BLK)],
                                    data_g.at[pl.ds(t * ROWS_BLK, ROWS_BLK)],
                                    gsem).wait()

                        @pl.when(j0 + GB <= _JP)
                        def _():
                            pltpu.sync_copy(
                                data_g,
                                _gout.at[pl.ds(j0 * ROWS_BLK, GROUP_ROWS),
                                         pl.ds(_col, FC)])

                        @pl.when(j0 + GB > _JP)
                        def _():
                            for t in range(GB):
                                j = j0 + t

                                @pl.when(j < _JP)
                                def _(t=t, j=j):
                                    pltpu.sync_copy(
                                        data_g.at[pl.ds(t * ROWS_BLK, ROWS_BLK)],
                                        _gout.at[pl.ds(j * ROWS_BLK, ROWS_BLK),
                                                 pl.ds(_col, FC)])
                                if _rem:
                                    @pl.when(j == _JP)
                                    def _(t=t):
                                        pltpu.sync_copy(
                                            data_g.at[pl.ds(t * ROWS_BLK, _rem)],
                                            _gout.at[pl.ds(_JP * ROWS_BLK, _rem),
                                                     pl.ds(_col, FC)])

            plsc.subcore_barrier()


def _make_sc_call():
    out_type = tuple(jax.ShapeDtypeStruct((r, H), jnp.float32)
                     for _, r in CYC + PTH)
    mesh = plsc.VectorSubcoreMesh(core_axis_name="c", subcore_axis_name="s")
    return pl.kernel(
        _sc_transfer,
        out_type=out_type,
        mesh=mesh,
        compiler_params=pltpu.CompilerParams(use_tc_tiling_on_sc=False),
        scratch_types=[
            pltpu.VMEM_SHARED((N_NODES, FC), jnp.float32),
            pltpu.VMEM((GB, ROWS_BLK), jnp.int32),
            pltpu.VMEM((GROUP_ROWS, FC), jnp.float32),
            pltpu.SemaphoreType.DMA,
            pltpu.SemaphoreType.DMA,
            pltpu.SemaphoreType.DMA,
        ],
    )


IB = 80


def _autobahn_body(s, g_ref, w1_ref, w0_ref, b_ref, o_ref):
    br = IB * s
    g = g_ref[...].astype(jnp.bfloat16)
    out1 = jnp.dot(g, w1_ref[...], preferred_element_type=jnp.float32)
    row_inst = lax.broadcasted_iota(jnp.int32, (IB, br), 1) // s
    inst = lax.broadcasted_iota(jnp.int32, (IB, br), 0)
    m = (row_inst == inst).astype(jnp.bfloat16)
    tsum = jnp.dot(m, g, preferred_element_type=jnp.float32)
    t0 = jnp.dot(tsum.astype(jnp.bfloat16), w0_ref[...],
                 preferred_element_type=jnp.float32)
    row_inst_t = lax.broadcasted_iota(jnp.int32, (br, IB), 0) // s
    inst_t = lax.broadcasted_iota(jnp.int32, (br, IB), 1)
    mt = (row_inst_t == inst_t).astype(jnp.bfloat16)
    out0 = jnp.dot(mt, t0.astype(jnp.bfloat16),
                   preferred_element_type=jnp.float32)
    o_ref[...] = out1 + out0 + b_ref[...]


def _autobahn(s, rows, g, w, b):
    br = IB * s
    grid = rows // br
    w1 = w[:H].astype(jnp.bfloat16)
    w0 = w[H:].astype(jnp.bfloat16)
    b2 = b.reshape(1, H)
    return pl.pallas_call(
        functools.partial(_autobahn_body, s),
        grid=(grid,),
        in_specs=[
            pl.BlockSpec((br, H), lambda i: (i, 0)),
            pl.BlockSpec((H, H), lambda i: (0, 0)),
            pl.BlockSpec((H, H), lambda i: (0, 0)),
            pl.BlockSpec((1, H), lambda i: (0, 0)),
        ],
        out_specs=pl.BlockSpec((br, H), lambda i: (i, 0)),
        out_shape=jax.ShapeDtypeStruct((rows, H), jnp.float32),
        compiler_params=pltpu.CompilerParams(
            dimension_semantics=("arbitrary",)),
    )(g, w1, w0, b2)


def _pad_idx(nodes, r):
    p = _padded(r)
    pad = jnp.arange(p - r, dtype=jnp.int32) % N_NODES
    return jnp.concatenate([nodes, pad]).reshape(p // ROWS_BLK, ROWS_BLK)


def kernel(cycle_rep_5, cycle_nodes_5, cycle_rep_6, cycle_nodes_6,
           path_rep_4, path_nodes_4, path_rep_5, path_nodes_5,
           path_rep_6, path_nodes_6, path_rep_7, path_nodes_7,
           W_c5, b_c5, W_c6, b_c6,
           W_p4, b_p4, W_p5, b_p5, W_p6, b_p6, W_p7, b_p7):
    idx = [_pad_idx(n, r) for n, (_, r) in zip(
        (cycle_nodes_5, cycle_nodes_6, path_nodes_4, path_nodes_5,
         path_nodes_6, path_nodes_7), CYC + PTH)]
    zeros = jnp.zeros((1024, FC), jnp.float32)

    g_c5, g_c6, g_p4, g_p5, g_p6, g_p7 = _make_sc_call()(
        cycle_rep_5, cycle_rep_6, path_rep_4, path_rep_5, path_rep_6,
        path_rep_7, *idx, zeros)

    cycle_outs = (
        _autobahn(5, 50000, g_c5, W_c5, b_c5),
        _autobahn(6, 60000, g_c6, W_c6, b_c6),
    )
    path_outs = (
        _autobahn(4, 40000, g_p4, W_p4, b_p4),
        _autobahn(5, 50000, g_p5, W_p5, b_p5),
        _autobahn(6, 60000, g_p6, W_p6, b_p6),
        _autobahn(7, 70000, g_p7, W_p7, b_p7),
    )
    return (cycle_outs, path_outs)


# speedup vs baseline: 6.5660x; 1.1588x over previous
import functools

import jax
import jax.numpy as jnp
from jax import lax
from jax.experimental import pallas as pl
from jax.experimental.pallas import tpu as pltpu
from jax.experimental.pallas import tpu_sc as plsc

N_NODES = 50000
DUMP = 128
H = 128
FC = 32
GB = 3
ROWS_BLK = 128
GROUP_ROWS = GB * ROWS_BLK
TILE_PAD = 16 * ROWS_BLK * GB
CYC = ((5, 50000), (6, 60000))
PTH = ((4, 40000), (5, 50000), (6, 60000), (7, 70000))
NODES_PER_TILE = N_NODES // 16


def _padded(r):
    return ((r + TILE_PAD - 1) // TILE_PAD) * TILE_PAD


def _dump_ids(n, off):
    return (N_NODES + (jnp.arange(n, dtype=jnp.int32) + off) % DUMP)


def _scatter_idx(nodes, r):
    p = _padded(r)
    jp, rem = r // ROWS_BLK, r % ROWS_BLK
    parts = [nodes[:jp * ROWS_BLK]]
    if rem:
        parts += [_dump_ids(ROWS_BLK - rem, 0), nodes[jp * ROWS_BLK:]]
    nbpad = p // ROWS_BLK - jp - (1 if rem else 0)
    if nbpad:
        parts.append(_dump_ids(nbpad * ROWS_BLK, 7))
    return jnp.concatenate(parts).reshape(p // ROWS_BLK, ROWS_BLK)


def _gather_idx(nodes, r):
    p = _padded(r)
    jp = r // ROWS_BLK
    parts = [nodes[:jp * ROWS_BLK]] + [nodes[r - ROWS_BLK:]] * (p // ROWS_BLK - jp)
    return jnp.concatenate(parts).reshape(p // ROWS_BLK, ROWS_BLK)


def _mk_body(src_sizes, dst_sizes):
    n_src, n_dst = len(src_sizes), len(dst_sizes)

    def body(*refs):
        reps = refs[:n_src]
        sidxs = refs[n_src:2 * n_src]
        gidxs = refs[2 * n_src:2 * n_src + n_dst]
        zeros_hbm = refs[2 * n_src + n_dst]
        gouts = refs[2 * n_src + n_dst + 1:2 * n_src + n_dst + 1 + n_dst]
        table, idx2, data2, fsem, ssem, gsem, wsem = refs[-7:]

        cid = lax.axis_index("c")
        sid = lax.axis_index("s")

        def dslice(buf, t):
            return data2.at[pl.ds(pl.multiple_of(
                buf * GROUP_ROWS + t * ROWS_BLK, ROWS_BLK), ROWS_BLK)]

        def dfull(buf):
            return data2.at[pl.ds(pl.multiple_of(
                buf * GROUP_ROWS, ROWS_BLK), GROUP_ROWS)]

        def fill(buf, j0, rep, sidx, col, R):
            pltpu.async_copy(sidx.at[pl.ds(j0, GB)],
                             idx2.at[pl.ds(buf * GB, GB)], fsem.at[buf])
            for t in range(GB):
                o = pl.multiple_of(
                    jnp.minimum((j0 + t) * ROWS_BLK, R - ROWS_BLK), 16)
                pltpu.async_copy(
                    rep.at[pl.ds(o, ROWS_BLK), pl.ds(col, FC)],
                    dslice(buf, t), fsem.at[buf])

        def wait_fill(buf, sidx):
            pltpu.make_async_copy(sidx.at[pl.ds(0, GB)],
                                  idx2.at[pl.ds(0, GB)], fsem.at[buf]).wait()
            pltpu.make_async_copy(
                zeros_hbm, dfull(buf), fsem.at[buf]).wait()

        def drain3(sem, buf):
            for t in range(GB):
                pltpu.make_async_copy(
                    zeros_hbm.at[pl.ds(0, ROWS_BLK)],
                    dslice(buf, t), sem.at[buf]).wait()

        @pl.loop(0, 2)
        def _u(u):
            col = (2 * cid + u) * FC
            zbase = sid * NODES_PER_TILE
            pltpu.sync_copy(zeros_hbm, data2.at[pl.ds(0, GROUP_ROWS)])
            for t in range(8):
                pltpu.sync_copy(
                    data2.at[pl.ds(0, GROUP_ROWS)],
                    table.at[pl.ds(zbase + t * GROUP_ROWS, GROUP_ROWS)])
            pltpu.sync_copy(
                data2.at[pl.ds(0, NODES_PER_TILE - 8 * GROUP_ROWS)],
                table.at[pl.ds(zbase + 8 * GROUP_ROWS,
                               NODES_PER_TILE - 8 * GROUP_ROWS)])
            plsc.subcore_barrier()

            for a in range(n_src):
                rep, sidx, R = reps[a], sidxs[a], src_sizes[a]
                NBT = sidx.shape[0] // 16
                NG = NBT // GB
                base = sid * NBT
                fill(0, base, rep, sidx, col, R)

                @pl.loop(0, NG)
                def _g(g, rep=rep, sidx=sidx, R=R, NBT=NBT, NG=NG, base=base,
                       col=col):
                    b = g & 1
                    wait_fill(b, sidx)
                    for t in range(GB):
                        pltpu.async_copy(
                            dslice(b, t),
                            table.at[idx2.at[b * GB + t]], ssem.at[b],
                            add=True)

                    @pl.when(g >= 1)
                    def _():
                        drain3(ssem, 1 - b)

                    @pl.when(g + 1 < NG)
                    def _():
                        fill(1 - b, base + (g + 1) * GB, rep, sidx, col, R)

                drain3(ssem, (NG - 1) & 1)
            plsc.subcore_barrier()

            for a in range(n_dst):
                gout, gidx, R = gouts[a], gidxs[a], dst_sizes[a]
                NBT = gidx.shape[0] // 16
                NG = NBT // GB
                base = sid * NBT
                pltpu.async_copy(gidx.at[pl.ds(base, GB)],
                                 idx2.at[pl.ds(0, GB)], fsem.at[0])

                @pl.loop(0, NG)
                def _g(g, gout=gout, gidx=gidx, R=R, NBT=NBT, NG=NG,
                       base=base, col=col):
                    b = g & 1
                    pltpu.make_async_copy(gidx.at[pl.ds(0, GB)],
                                          idx2.at[pl.ds(0, GB)],
                                          fsem.at[b]).wait()
                    for t in range(GB):
                        pltpu.async_copy(
                            table.at[idx2.at[b * GB + t]],
                            dslice(b, t), gsem.at[b])

                    @pl.when(g + 1 < NG)
                    def _():
                        pltpu.async_copy(
                            gidx.at[pl.ds(base + (g + 1) * GB, GB)],
                            idx2.at[pl.ds((1 - b) * GB, GB)],
                            fsem.at[1 - b])

                    drain3(gsem, b)

                    @pl.when(g >= 1)
                    def _():
                        drain3(wsem, 1 - b)

                    j0 = base + g * GB
                    for t in range(GB):
                        o = pl.multiple_of(
                            jnp.minimum((j0 + t) * ROWS_BLK, R - ROWS_BLK), 16)
                        pltpu.async_copy(
                            dslice(b, t),
                            gout.at[pl.ds(o, ROWS_BLK), pl.ds(col, FC)],
                            wsem.at[b])

                drain3(wsem, (NG - 1) & 1)
            plsc.subcore_barrier()

    return body


def _make_sc_call(src_sizes, dst_sizes):
    out_type = tuple(jax.ShapeDtypeStruct((r, H), jnp.float32)
                     for r in dst_sizes)
    mesh = plsc.VectorSubcoreMesh(core_axis_name="c", subcore_axis_name="s")
    return pl.kernel(
        _mk_body(src_sizes, dst_sizes),
        out_type=out_type,
        mesh=mesh,
        compiler_params=pltpu.CompilerParams(use_tc_tiling_on_sc=False),
        scratch_types=[
            pltpu.VMEM_SHARED((N_NODES + DUMP, FC), jnp.float32),
            pltpu.VMEM((2 * GB, ROWS_BLK), jnp.int32),
            pltpu.VMEM((2 * GROUP_ROWS, FC), jnp.float32),
            pltpu.SemaphoreType.DMA((2,)),
            pltpu.SemaphoreType.DMA((2,)),
            pltpu.SemaphoreType.DMA((2,)),
            pltpu.SemaphoreType.DMA((2,)),
        ],
    )


IB = 80


def _autobahn_body(s, g_ref, w1_ref, w0_ref, b_ref, o_ref):
    br = IB * s
    g = g_ref[...].astype(jnp.bfloat16)
    out1 = jnp.dot(g, w1_ref[...], preferred_element_type=jnp.float32)
    row_inst = lax.broadcasted_iota(jnp.int32, (IB, br), 1) // s
    inst = lax.broadcasted_iota(jnp.int32, (IB, br), 0)
    m = (row_inst == inst).astype(jnp.bfloat16)
    tsum = jnp.dot(m, g, preferred_element_type=jnp.float32)
    t0 = jnp.dot(tsum.astype(jnp.bfloat16), w0_ref[...],
                 preferred_element_type=jnp.float32)
    row_inst_t = lax.broadcasted_iota(jnp.int32, (br, IB), 0) // s
    inst_t = lax.broadcasted_iota(jnp.int32, (br, IB), 1)
    mt = (row_inst_t == inst_t).astype(jnp.bfloat16)
    out0 = jnp.dot(mt, t0.astype(jnp.bfloat16),
                   preferred_element_type=jnp.float32)
    o_ref[...] = out1 + out0 + b_ref[...]


def _autobahn(s, rows, g, w, b):
    br = IB * s
    grid = rows // br
    w1 = w[:H].astype(jnp.bfloat16)
    w0 = w[H:].astype(jnp.bfloat16)
    b2 = b.reshape(1, H)
    return pl.pallas_call(
        functools.partial(_autobahn_body, s),
        grid=(grid,),
        in_specs=[
            pl.BlockSpec((br, H), lambda i: (i, 0)),
            pl.BlockSpec((H, H), lambda i: (0, 0)),
            pl.BlockSpec((H, H), lambda i: (0, 0)),
            pl.BlockSpec((1, H), lambda i: (0, 0)),
        ],
        out_specs=pl.BlockSpec((br, H), lambda i: (i, 0)),
        out_shape=jax.ShapeDtypeStruct((rows, H), jnp.float32),
        compiler_params=pltpu.CompilerParams(
            dimension_semantics=("arbitrary",)),
    )(g, w1, w0, b2)


def kernel(cycle_rep_5, cycle_nodes_5, cycle_rep_6, cycle_nodes_6,
           path_rep_4, path_nodes_4, path_rep_5, path_nodes_5,
           path_rep_6, path_nodes_6, path_rep_7, path_nodes_7,
           W_c5, b_c5, W_c6, b_c6,
           W_p4, b_p4, W_p5, b_p5, W_p6, b_p6, W_p7, b_p7):
    c_nodes = (cycle_nodes_5, cycle_nodes_6)
    p_nodes = (path_nodes_4, path_nodes_5, path_nodes_6, path_nodes_7)
    c_reps = (cycle_rep_5, cycle_rep_6)
    p_reps = (path_rep_4, path_rep_5, path_rep_6, path_rep_7)
    c_sizes = tuple(r for _, r in CYC)
    p_sizes = tuple(r for _, r in PTH)

    c_sidx = [_scatter_idx(n, r) for n, r in zip(c_nodes, c_sizes)]
    p_sidx = [_scatter_idx(n, r) for n, r in zip(p_nodes, p_sizes)]
    c_gidx = [_gather_idx(n, r) for n, r in zip(c_nodes, c_sizes)]
    p_gidx = [_gather_idx(n, r) for n, r in zip(p_nodes, p_sizes)]
    zeros = jnp.zeros((GROUP_ROWS, FC), jnp.float32)

    g_c5, g_c6 = _make_sc_call(p_sizes, c_sizes)(
        *p_reps, *p_sidx, *c_gidx, zeros)
    g_p4, g_p5, g_p6, g_p7 = _make_sc_call(c_sizes, p_sizes)(
        *c_reps, *c_sidx, *p_gidx, zeros)

    cycle_outs = (
        _autobahn(5, 50000, g_c5, W_c5, b_c5),
        _autobahn(6, 60000, g_c6, W_c6, b_c6),
    )
    path_outs = (
        _autobahn(4, 40000, g_p4, W_p4, b_p4),
        _autobahn(5, 50000, g_p5, W_p5, b_p5),
        _autobahn(6, 60000, g_p6, W_p6, b_p6),
        _autobahn(7, 70000, g_p7, W_p7, b_p7),
    )
    return (cycle_outs, path_outs)
